# baseline (speedup 1.0000x reference)
import functools

import jax
import jax.numpy as jnp
from jax import lax
from jax.experimental import pallas as pl
from jax.experimental.pallas import tpu as pltpu
from jax.experimental.pallas import tpu_sc as plsc

DIM = 33
NLUT = 3 * DIM ** 3
NC, NS, L = 2, 16, 16
NW = NC * NS
CHUNK = 512


def kernel(LUT, x):
    B, C, W, H = x.shape
    P = W * H
    N = B * P
    per_w = N // NW
    n_chunks = per_w // CHUNK
    wpb = P // per_w

    x_flat = x.reshape(B * C, P)
    lut_flat = LUT.reshape(NLUT)
    inv_binsize = jnp.float32((DIM - 1) / 1.000001)

    mesh = plsc.VectorSubcoreMesh(
        core_axis_name="c", subcore_axis_name="s", num_cores=NC, num_subcores=NS
    )

    buf = lambda: pltpu.VMEM((CHUNK,), jnp.float32)

    @functools.partial(
        pl.kernel,
        out_type=jax.ShapeDtypeStruct((B * C, P), jnp.float32),
        mesh=mesh,
        compiler_params=pltpu.CompilerParams(needs_layout_passes=False),
        scratch_types=(
            [pltpu.VMEM((NLUT,), jnp.float32)]
            + [buf() for _ in range(12)]
            + [pltpu.SemaphoreType.DMA for _ in range(4)]
        ),
    )
    def lut_kernel(lut_hbm, x_hbm, out_hbm, lut_v, *rest):
        ins = ((rest[0], rest[1], rest[2]), (rest[3], rest[4], rest[5]))
        outs = ((rest[6], rest[7], rest[8]), (rest[9], rest[10], rest[11]))
        sem_in = (rest[12], rest[13])
        sem_out = (rest[14], rest[15])

        wid = lax.axis_index("s") * NC + lax.axis_index("c")
        batch = wid // wpb
        base_px = (wid % wpb) * per_w
        row0 = 3 * batch

        pltpu.sync_copy(lut_hbm, lut_v)

        def issue_in(j, p):
            start = base_px + j * CHUNK
            for c in range(3):
                pltpu.async_copy(
                    x_hbm.at[row0 + c, pl.ds(start, CHUNK)], ins[p][c], sem_in[p]
                )

        def drain_in(p):
            for c in range(3):
                pltpu.make_async_copy(
                    x_hbm.at[row0, pl.ds(0, CHUNK)], ins[p][c], sem_in[p]
                ).wait()

        def issue_out(j, p):
            start = base_px + j * CHUNK
            for c in range(3):
                pltpu.async_copy(
                    outs[p][c], out_hbm.at[row0 + c, pl.ds(start, CHUNK)], sem_out[p]
                )

        def drain_out(p):
            for c in range(3):
                pltpu.make_async_copy(
                    x_hbm.at[row0, pl.ds(0, CHUNK)], outs[p][c], sem_out[p]
                ).wait()

        offs = (0, 1, DIM, DIM + 1,
                DIM * DIM, DIM * DIM + 1, DIM * DIM + DIM, DIM * DIM + DIM + 1)

        def corner_ref(c, k):
            o = (c * (DIM ** 3) + offs[k]) & ~7
            return lut_v.at[pl.ds(o, NLUT - o)]

        def corner_rem(c, k):
            return (c * (DIM ** 3) + offs[k]) & 7

        def compute(p):
            @plsc.parallel_loop(0, CHUNK, L, unroll=2)
            def px_body(off):
                r = ins[p][0][pl.ds(off, L)]
                g = ins[p][1][pl.ds(off, L)]
                b = ins[p][2][pl.ds(off, L)]
                rs = r * inv_binsize
                gs = g * inv_binsize
                bs = b * inv_binsize
                rid = rs.astype(jnp.int32)
                gid = gs.astype(jnp.int32)
                bid = bs.astype(jnp.int32)
                rd = rs - rid.astype(jnp.float32)
                gd = gs - gid.astype(jnp.float32)
                bd = bs - bid.astype(jnp.float32)
                base = rid + gid * DIM + bid * (DIM * DIM)

                ar = 1.0 - rd
                ag = 1.0 - gd
                ab = 1.0 - bd
                p00 = ag * ab
                p10 = gd * ab
                p01 = ag * bd
                p11 = gd * bd
                w = (ar * p00, rd * p00, ar * p10, rd * p10,
                     ar * p01, rd * p01, ar * p11, rd * p11)
                bases = [base]
                for r in range(1, 6):
                    bases.append(bases[-1] + 1)
                for c in range(3):
                    ps = [w[k] * plsc.load_gather(corner_ref(c, k), [bases[corner_rem(c, k)]])
                          for k in range(8)]
                    s0 = (ps[0] + ps[1]) + (ps[2] + ps[3])
                    s1 = (ps[4] + ps[5]) + (ps[6] + ps[7])
                    outs[p][c][pl.ds(off, L)] = s0 + s1

        issue_in(0, 0)
        issue_in(1, 1)

        def pair_body(t, _):
            j = 2 * t
            for p in range(2):
                jj = j + p
                drain_in(p)

                @pl.when(jj >= 2)
                def _():
                    drain_out(p)

                compute(p)
                issue_out(jj, p)

                @pl.when(jj + 2 < n_chunks)
                def _():
                    issue_in(jj + 2, p)

            return 0

        lax.fori_loop(0, n_chunks // 2, pair_body, 0)
        drain_out(0)
        drain_out(1)

    out = lut_kernel(lut_flat, x_flat)
    return out.reshape(B, C, W, H)


# speedup vs baseline: 1.0185x; 1.0185x over previous
import functools

import jax
import jax.numpy as jnp
from jax import lax
from jax.experimental import pallas as pl
from jax.experimental.pallas import tpu as pltpu
from jax.experimental.pallas import tpu_sc as plsc

DIM = 33
NLUT = 3 * DIM ** 3
NC, NS, L = 2, 16, 16
NW = NC * NS
CHUNK = 1024


def kernel(LUT, x):
    B, C, W, H = x.shape
    P = W * H
    N = B * P
    per_w = N // NW
    n_chunks = per_w // CHUNK
    wpb = P // per_w

    x_flat = x.reshape(B * C, P)
    lut2 = LUT.reshape(3, DIM ** 3)
    lo = lax.bitcast_convert_type(lut2.astype(jnp.bfloat16), jnp.uint16)
    hi = jnp.concatenate(
        [lo[:, 1:], jnp.zeros((3, 1), jnp.uint16)], axis=1)
    packed = lo.astype(jnp.uint32) | (hi.astype(jnp.uint32) << 16)
    lut_flat = lax.bitcast_convert_type(packed, jnp.int32).reshape(NLUT)
    inv_binsize = jnp.float32((DIM - 1) / 1.000001)

    mesh = plsc.VectorSubcoreMesh(
        core_axis_name="c", subcore_axis_name="s", num_cores=NC, num_subcores=NS
    )

    buf = lambda: pltpu.VMEM((CHUNK,), jnp.float32)

    @functools.partial(
        pl.kernel,
        out_type=jax.ShapeDtypeStruct((B * C, P), jnp.float32),
        mesh=mesh,
        compiler_params=pltpu.CompilerParams(needs_layout_passes=False),
        scratch_types=(
            [pltpu.VMEM((NLUT,), jnp.int32)]
            + [buf() for _ in range(12)]
            + [pltpu.SemaphoreType.DMA for _ in range(4)]
        ),
    )
    def lut_kernel(lut_hbm, x_hbm, out_hbm, lut_v, *rest):
        ins = ((rest[0], rest[1], rest[2]), (rest[3], rest[4], rest[5]))
        outs = ((rest[6], rest[7], rest[8]), (rest[9], rest[10], rest[11]))
        sem_in = (rest[12], rest[13])
        sem_out = (rest[14], rest[15])

        wid = lax.axis_index("s") * NC + lax.axis_index("c")
        batch = wid // wpb
        base_px = (wid % wpb) * per_w
        row0 = 3 * batch

        pltpu.sync_copy(lut_hbm, lut_v)

        def issue_in(j, p):
            start = base_px + j * CHUNK
            for c in range(3):
                pltpu.async_copy(
                    x_hbm.at[row0 + c, pl.ds(start, CHUNK)], ins[p][c], sem_in[p]
                )

        def drain_in(p):
            for c in range(3):
                pltpu.make_async_copy(
                    x_hbm.at[row0, pl.ds(0, CHUNK)], ins[p][c], sem_in[p]
                ).wait()

        def issue_out(j, p):
            start = base_px + j * CHUNK
            for c in range(3):
                pltpu.async_copy(
                    outs[p][c], out_hbm.at[row0 + c, pl.ds(start, CHUNK)], sem_out[p]
                )

        def drain_out(p):
            for c in range(3):
                pltpu.make_async_copy(
                    x_hbm.at[row0, pl.ds(0, CHUNK)], outs[p][c], sem_out[p]
                ).wait()

        offs = (0, DIM, DIM * DIM, DIM * DIM + DIM)

        def corner_ref(c, k):
            o = (c * (DIM ** 3) + offs[k]) & ~7
            return lut_v.at[pl.ds(o, NLUT - o)]

        def corner_rem(c, k):
            return (c * (DIM ** 3) + offs[k]) & 7

        def compute(p):
            @plsc.parallel_loop(0, CHUNK, L, unroll=2)
            def px_body(off):
                r = ins[p][0][pl.ds(off, L)]
                g = ins[p][1][pl.ds(off, L)]
                b = ins[p][2][pl.ds(off, L)]
                rs = r * inv_binsize
                gs = g * inv_binsize
                bs = b * inv_binsize
                rid = rs.astype(jnp.int32)
                gid = gs.astype(jnp.int32)
                bid = bs.astype(jnp.int32)
                rd = rs - rid.astype(jnp.float32)
                gd = gs - gid.astype(jnp.float32)
                bd = bs - bid.astype(jnp.float32)
                base = rid + gid * DIM + bid * (DIM * DIM)

                ar = 1.0 - rd
                ag = 1.0 - gd
                ab = 1.0 - bd
                p00 = ag * ab
                p10 = gd * ab
                p01 = ag * bd
                p11 = gd * bd
                w = (ar * p00, rd * p00, ar * p10, rd * p10,
                     ar * p01, rd * p01, ar * p11, rd * p11)
                bases = [base]
                for r in range(1, 5):
                    bases.append(bases[-1] + 1)
                himask = jnp.int32(-65536)
                for c in range(3):
                    acc = None
                    for k in range(4):
                        v = plsc.load_gather(corner_ref(c, k), [bases[corner_rem(c, k)]])
                        lo_f = lax.bitcast_convert_type(v << 16, jnp.float32)
                        hi_f = lax.bitcast_convert_type(v & himask, jnp.float32)
                        term = w[2 * k] * lo_f + w[2 * k + 1] * hi_f
                        acc = term if acc is None else acc + term
                    outs[p][c][pl.ds(off, L)] = acc

        issue_in(0, 0)
        issue_in(1, 1)

        def pair_body(t, _):
            j = 2 * t
            for p in range(2):
                jj = j + p
                drain_in(p)

                @pl.when(jj >= 2)
                def _():
                    drain_out(p)

                compute(p)
                issue_out(jj, p)

                @pl.when(jj + 2 < n_chunks)
                def _():
                    issue_in(jj + 2, p)

            return 0

        lax.fori_loop(0, n_chunks // 2, pair_body, 0)
        drain_out(0)
        drain_out(1)

    out = lut_kernel(lut_flat, x_flat)
    return out.reshape(B, C, W, H)


# speedup vs baseline: 1.0466x; 1.0276x over previous
import functools

import jax
import jax.numpy as jnp
from jax import lax
from jax.experimental import pallas as pl
from jax.experimental.pallas import tpu as pltpu
from jax.experimental.pallas import tpu_sc as plsc

DIM = 33
NLUT = 3 * DIM ** 3
NC, NS, L = 2, 16, 16
NW = NC * NS
CHUNK = 1024


def kernel(LUT, x):
    B, C, W, H = x.shape
    P = W * H
    N = B * P
    per_w = N // NW
    n_chunks = per_w // CHUNK
    wpb = P // per_w

    x_flat = x.reshape(B * C, P)
    lut2 = LUT.reshape(3, DIM ** 3)
    lo = lax.bitcast_convert_type(lut2.astype(jnp.bfloat16), jnp.uint16)
    hi = jnp.concatenate(
        [lo[:, 1:], jnp.zeros((3, 1), jnp.uint16)], axis=1)
    packed = lo.astype(jnp.uint32) | (hi.astype(jnp.uint32) << 16)
    lut_flat = lax.bitcast_convert_type(packed, jnp.int32).reshape(NLUT)
    inv_binsize = jnp.float32((DIM - 1) / 1.000001)

    mesh = plsc.VectorSubcoreMesh(
        core_axis_name="c", subcore_axis_name="s", num_cores=NC, num_subcores=NS
    )

    buf = lambda: pltpu.VMEM((CHUNK,), jnp.float32)

    @functools.partial(
        pl.kernel,
        out_type=jax.ShapeDtypeStruct((B * C, P), jnp.float32),
        mesh=mesh,
        compiler_params=pltpu.CompilerParams(needs_layout_passes=False),
        scratch_types=(
            [pltpu.VMEM((NLUT,), jnp.int32)]
            + [buf() for _ in range(12)]
            + [pltpu.SemaphoreType.DMA for _ in range(4)]
        ),
    )
    def lut_kernel(lut_hbm, x_hbm, out_hbm, lut_v, *rest):
        ins = ((rest[0], rest[1], rest[2]), (rest[3], rest[4], rest[5]))
        outs = ((rest[6], rest[7], rest[8]), (rest[9], rest[10], rest[11]))
        sem_in = (rest[12], rest[13])
        sem_out = (rest[14], rest[15])

        wid = lax.axis_index("s") * NC + lax.axis_index("c")
        batch = wid // wpb
        base_px = (wid % wpb) * per_w
        row0 = 3 * batch

        pltpu.sync_copy(lut_hbm, lut_v)

        def issue_in(j, p):
            start = base_px + j * CHUNK
            for c in range(3):
                pltpu.async_copy(
                    x_hbm.at[row0 + c, pl.ds(start, CHUNK)], ins[p][c], sem_in[p]
                )

        def drain_in(p):
            for c in range(3):
                pltpu.make_async_copy(
                    x_hbm.at[row0, pl.ds(0, CHUNK)], ins[p][c], sem_in[p]
                ).wait()

        def issue_out(j, p):
            start = base_px + j * CHUNK
            for c in range(3):
                pltpu.async_copy(
                    outs[p][c], out_hbm.at[row0 + c, pl.ds(start, CHUNK)], sem_out[p]
                )

        def drain_out(p):
            for c in range(3):
                pltpu.make_async_copy(
                    x_hbm.at[row0, pl.ds(0, CHUNK)], outs[p][c], sem_out[p]
                ).wait()

        offs = (0, DIM, DIM * DIM, DIM * DIM + DIM)

        def corner_ref(c, k):
            o = (c * (DIM ** 3) + offs[k]) & ~7
            return lut_v.at[pl.ds(o, NLUT - o)]

        def corner_rem(c, k):
            return (c * (DIM ** 3) + offs[k]) & 7

        def compute(p):
            @plsc.parallel_loop(0, CHUNK, L, unroll=3)
            def px_body(off):
                r = ins[p][0][pl.ds(off, L)]
                g = ins[p][1][pl.ds(off, L)]
                b = ins[p][2][pl.ds(off, L)]
                rs = r * inv_binsize
                gs = g * inv_binsize
                bs = b * inv_binsize
                rid = rs.astype(jnp.int32)
                gid = gs.astype(jnp.int32)
                bid = bs.astype(jnp.int32)
                rd = rs - rid.astype(jnp.float32)
                gd = gs - gid.astype(jnp.float32)
                bd = bs - bid.astype(jnp.float32)
                base = rid + gid * DIM + bid * (DIM * DIM)

                ar = 1.0 - rd
                ag = 1.0 - gd
                ab = 1.0 - bd
                p00 = ag * ab
                p10 = gd * ab
                p01 = ag * bd
                p11 = gd * bd
                w = (ar * p00, rd * p00, ar * p10, rd * p10,
                     ar * p01, rd * p01, ar * p11, rd * p11)
                bases = [base]
                for r in range(1, 5):
                    bases.append(bases[-1] + 1)
                himask = jnp.int32(-65536)
                for c in range(3):
                    acc = None
                    for k in range(4):
                        v = plsc.load_gather(corner_ref(c, k), [bases[corner_rem(c, k)]])
                        lo_f = lax.bitcast_convert_type(v << 16, jnp.float32)
                        hi_f = lax.bitcast_convert_type(v & himask, jnp.float32)
                        term = w[2 * k] * lo_f + w[2 * k + 1] * hi_f
                        acc = term if acc is None else acc + term
                    outs[p][c][pl.ds(off, L)] = acc

        issue_in(0, 0)
        issue_in(1, 1)

        def pair_body(t, _):
            j = 2 * t
            for p in range(2):
                jj = j + p
                drain_in(p)

                @pl.when(jj >= 2)
                def _():
                    drain_out(p)

                compute(p)
                issue_out(jj, p)

                @pl.when(jj + 2 < n_chunks)
                def _():
                    issue_in(jj + 2, p)

            return 0

        lax.fori_loop(0, n_chunks // 2, pair_body, 0)
        drain_out(0)
        drain_out(1)

    out = lut_kernel(lut_flat, x_flat)
    return out.reshape(B, C, W, H)
